# Initial kernel scaffold; baseline (speedup 1.0000x reference)
import jax
import jax.numpy as jnp
from jax.experimental import pallas as pl


def kernel(x, edge_index, edge_attr, W_msg, b_msg, W_ih, W_hh, b_ih, b_hh):
    raise NotImplementedError("write your pallas kernel here")


# speedup vs baseline: 5.3970x; 5.3970x over previous
import functools

import jax
import jax.numpy as jnp
from jax import lax
from jax.experimental import pallas as pl
from jax.experimental.pallas import tpu as pltpu
from jax.experimental.pallas import tpu_sc as plsc

N = 10000
E = 320000
H = 128
FE = 16

def _rne_bf16(x):
    u = jax.lax.bitcast_convert_type(x, jnp.uint32)
    r = (u + jnp.uint32(0x7FFF) + ((u >> 16) & jnp.uint32(1))) \
        & jnp.uint32(0xFFFF0000)
    return jax.lax.bitcast_convert_type(r, jnp.float32)


NC = 2
NS = 16
NW = NC * NS
CHUNK = 128
STEPS = -(-E // (NW * CHUNK))
EPAD = NW * STEPS * CHUNK
NP = NS * 640
RPT = NP // NS
AW = 128


def _seg_sum_h(h, src3, dst3, zeros_h):
    mesh = plsc.VectorSubcoreMesh(core_axis_name="c", subcore_axis_name="s")

    @functools.partial(
        pl.kernel,
        out_type=jax.ShapeDtypeStruct((NC, NP, H), jnp.float32),
        mesh=mesh,
        scratch_types=[
            pltpu.VMEM((CHUNK, H), jnp.float32),
            pltpu.VMEM((STEPS, CHUNK), jnp.int32),
            pltpu.VMEM((STEPS, CHUNK), jnp.int32),
            pltpu.VMEM_SHARED((NP, H), jnp.float32),
            pltpu.SemaphoreType.DMA,
        ],
    )
    def k(h_hbm, src_hbm, dst_hbm, z_hbm, out_hbm, rows_v, src_v, dst_v, acc_s, sem):
        c = lax.axis_index("c")
        s = lax.axis_index("s")
        wid = c * NS + s
        pltpu.sync_copy(z_hbm, acc_s.at[pl.ds(s * RPT, RPT)])
        pltpu.sync_copy(src_hbm.at[wid], src_v)
        pltpu.sync_copy(dst_hbm.at[wid], dst_v)
        plsc.subcore_barrier()

        def body(j, carry):
            pltpu.async_copy(h_hbm.at[src_v.at[j]], rows_v, sem).wait()
            pltpu.sync_copy(rows_v, acc_s.at[dst_v.at[j]], add=True)
            return carry

        lax.fori_loop(0, STEPS, body, 0)
        plsc.subcore_barrier()
        pltpu.sync_copy(acc_s.at[pl.ds(s * RPT, RPT)],
                        out_hbm.at[c, pl.ds(s * RPT, RPT)])

    return k(h, src3, dst3, zeros_h)


def _seg_sum_attr(attr4, dst3, zeros_a):
    mesh = plsc.VectorSubcoreMesh(core_axis_name="c", subcore_axis_name="s")

    @functools.partial(
        pl.kernel,
        out_type=jax.ShapeDtypeStruct((NC, NP, AW), jnp.float32),
        mesh=mesh,
        scratch_types=[
            pltpu.VMEM((CHUNK, AW), jnp.float32),
            pltpu.VMEM((STEPS, CHUNK), jnp.int32),
            pltpu.VMEM_SHARED((NP, AW), jnp.float32),
            pltpu.SemaphoreType.DMA,
        ],
    )
    def k(a_hbm, dst_hbm, z_hbm, out_hbm, rows_v, dst_v, acc_s, sem):
        c = lax.axis_index("c")
        s = lax.axis_index("s")
        wid = c * NS + s
        pltpu.sync_copy(z_hbm, acc_s.at[pl.ds(s * RPT, RPT)])
        pltpu.sync_copy(dst_hbm.at[wid], dst_v)
        plsc.subcore_barrier()

        def body(j, carry):
            pltpu.sync_copy(a_hbm.at[wid, j], rows_v)
            pltpu.sync_copy(rows_v, acc_s.at[dst_v.at[j]], add=True)
            return carry

        lax.fori_loop(0, STEPS, body, 0)
        plsc.subcore_barrier()
        pltpu.sync_copy(acc_s.at[pl.ds(s * RPT, RPT)],
                        out_hbm.at[c, pl.ds(s * RPT, RPT)])

    return k(attr4, dst3, zeros_a)


BN = 2000


def _dense_round_body(h_ref, p_ref, ead_ref, w1_ref, w2_ref, w3_ref, bm_ref,
                      wih_ref, whh_ref, bih_ref, bhh_ref, out_ref):
    hp = jax.lax.Precision.HIGHEST
    f32 = jnp.float32
    bf = jnp.bfloat16

    def bdot(a, b):
        return jnp.dot(a.astype(bf), b.astype(bf), preferred_element_type=f32)

    def rdot(a, b):
        return jnp.dot(a, b, precision=hp)

    h = h_ref[...]
    ssum = p_ref[0] + p_ref[1]
    ead = ead_ref[0] + ead_ref[1]
    ea = ead[:, :FE]
    deg = ead[:, FE:FE + 1]
    hw1 = bdot(h, w1_ref[...])
    a = (deg * (hw1 + bm_ref[...])
         + rdot(ssum, w2_ref[...])
         + rdot(ea, w3_ref[...]))
    gi = bdot(a, wih_ref[...]) + bih_ref[...]
    gh = bdot(h, whh_ref[...]) + bhh_ref[...]
    r = jax.nn.sigmoid(gi[:, :H] + gh[:, :H])
    z = jax.nn.sigmoid(gi[:, H:2 * H] + gh[:, H:2 * H])
    n = jnp.tanh(gi[:, 2 * H:] + r * gh[:, 2 * H:])
    out_ref[...] = (1.0 - z) * n + z * h


def _dense_round(h, parts, ead, w1, w2, w3, bm, wih, whh, bih, bhh):
    grid = N // BN
    return pl.pallas_call(
        _dense_round_body,
        grid=(grid,),
        in_specs=[
            pl.BlockSpec((BN, H), lambda i: (i, 0)),
            pl.BlockSpec((NC, BN, H), lambda i: (0, i, 0)),
            pl.BlockSpec((NC, BN, AW), lambda i: (0, i, 0)),
            pl.BlockSpec((H, 2 * H), lambda i: (0, 0)),
            pl.BlockSpec((H, 2 * H), lambda i: (0, 0)),
            pl.BlockSpec((FE, 2 * H), lambda i: (0, 0)),
            pl.BlockSpec((1, 2 * H), lambda i: (0, 0)),
            pl.BlockSpec((2 * H, 3 * H), lambda i: (0, 0)),
            pl.BlockSpec((H, 3 * H), lambda i: (0, 0)),
            pl.BlockSpec((1, 3 * H), lambda i: (0, 0)),
            pl.BlockSpec((1, 3 * H), lambda i: (0, 0)),
        ],
        out_specs=pl.BlockSpec((BN, H), lambda i: (i, 0)),
        out_shape=jax.ShapeDtypeStruct((N, H), jnp.float32),
    )(h, parts, ead, w1, w2, w3, bm, wih, whh, bih, bhh)


def kernel(x, edge_index, edge_attr, W_msg, b_msg, W_ih, W_hh, b_ih, b_hh):
    src = edge_index[0].astype(jnp.int32)
    dst = edge_index[1].astype(jnp.int32)
    pad = EPAD - E
    src3 = jnp.concatenate([src, jnp.zeros((pad,), jnp.int32)]
                           ).reshape(NW, STEPS, CHUNK)
    dst3 = jnp.concatenate([dst, jnp.full((pad,), N, jnp.int32)]
                           ).reshape(NW, STEPS, CHUNK)
    ones = jnp.ones((E, 1), jnp.float32)
    attr_r = _rne_bf16(edge_attr)
    attr = jnp.concatenate(
        [attr_r, ones, jnp.zeros((E, AW - FE - 1), jnp.float32)], axis=1)
    attr4 = jnp.concatenate([attr, jnp.zeros((pad, AW), jnp.float32)]
                            ).reshape(NW, STEPS, CHUNK, AW)
    zeros_h = jnp.zeros((RPT, H), jnp.float32)

    ead = _seg_sum_attr(attr4, dst3, zeros_h)
    dep = jnp.minimum(jnp.abs(ead[0, 0, 0]), 0.0)
    zeros_dep = zeros_h + dep

    h = x
    T = W_msg.shape[0]
    for t in range(T):
        hb = _rne_bf16(h)
        parts = _seg_sum_h(hb, src3, dst3, zeros_dep if t == 0 else zeros_h)
        w1 = W_msg[t, :H]
        w2 = _rne_bf16(W_msg[t, H:2 * H])
        w3 = _rne_bf16(W_msg[t, 2 * H:])
        h = _dense_round(h, parts, ead, w1, w2, w3,
                         b_msg[t].reshape(1, 2 * H), W_ih[t], W_hh[t],
                         b_ih[t].reshape(1, 3 * H), b_hh[t].reshape(1, 3 * H))
    return h
